# baseline (speedup 1.0000x reference)
import functools

import jax
import jax.numpy as jnp
from jax import lax
from jax.experimental import pallas as pl
from jax.experimental.pallas import tpu as pltpu
from jax.experimental.pallas import tpu_sc as plsc

VOCAB = 100000
D = 64
DP = 128
NC = 2
NS = 16
NW = NC * NS
BLK = 128


def _dense_body(emb_ref, w1_ref, b1_ref, w2_ref, b2_ref, out_ref):
    x = emb_ref[...]
    h1 = jnp.maximum(
        jnp.dot(x, w1_ref[...], preferred_element_type=jnp.float32) + b1_ref[...],
        0.0,
    )
    out_ref[...] = (
        jnp.dot(h1, w2_ref[...], preferred_element_type=jnp.float32) + b2_ref[...]
    )


def _dense_transform(emb, W1, b1, W2p, b2p):
    rows_per_block = 2000
    grid = VOCAB // rows_per_block
    return pl.pallas_call(
        _dense_body,
        grid=(grid,),
        in_specs=[
            pl.BlockSpec((rows_per_block, D), lambda i: (i, 0)),
            pl.BlockSpec((D, D), lambda i: (0, 0)),
            pl.BlockSpec((1, D), lambda i: (0, 0)),
            pl.BlockSpec((D, DP), lambda i: (0, 0)),
            pl.BlockSpec((1, DP), lambda i: (0, 0)),
        ],
        out_specs=pl.BlockSpec((rows_per_block, DP), lambda i: (i, 0)),
        out_shape=jax.ShapeDtypeStruct((VOCAB, DP), jnp.float32),
    )(emb, W1, b1.reshape(1, D), W2p, b2p.reshape(1, DP))


def _gather_t_body(h_hbm, idx_hbm, out_hbm, idx_v, buf_a, buf_b, tr_a, tr_b,
                   gsem, wsem):
    cid = lax.axis_index("c")
    sid = lax.axis_index("s")
    w = sid * NC + cid
    n_s = idx_hbm.shape[1]
    pltpu.sync_copy(idx_hbm.at[w], idx_v)
    col = w * BLK
    iota16 = lax.iota(jnp.int32, 16)
    row_idx = [iota16 + dc * 16 for dc in range(D // 16)]
    rots = [(iota16 + t) & 15 for t in range(16)]
    bufs = ((buf_a, tr_a), (buf_b, tr_b))

    def transpose_block(src, dst):
        @plsc.parallel_loop(0, BLK // 16, unroll=2)
        def _(jc):
            jc16 = jc * 16
            jvecs = [r + jc16 for r in rots]
            for dc in range(D // 16):
                dvec = row_idx[dc]
                for t in range(16):
                    v = plsc.load_gather(src, [jvecs[t], dvec])
                    plsc.store_scatter(dst, [dvec, jvecs[t]], v)

    def step(i, carry):
        descs = []
        for half in range(2):
            g = 2 * i + half
            buf, tr = bufs[half]

            @pl.when(i > 0)
            def _(tr=tr):
                pltpu.make_async_copy(
                    tr, out_hbm.at[0, :, pl.ds(col, BLK)], wsem
                ).wait()

            descs.append(pltpu.async_copy(h_hbm.at[idx_v.at[g]], buf, gsem))
        for half in range(2):
            g = 2 * i + half
            buf, tr = bufs[half]
            descs[half].wait()
            transpose_block(buf, tr)
            pltpu.async_copy(tr, out_hbm.at[g, :, pl.ds(col, BLK)], wsem)
        return carry

    lax.fori_loop(0, n_s // 2, step, 0)
    pltpu.make_async_copy(tr_a, out_hbm.at[0, :, pl.ds(col, BLK)], wsem).wait()
    pltpu.make_async_copy(tr_b, out_hbm.at[0, :, pl.ds(col, BLK)], wsem).wait()


def _gather_transpose(h_pad, idx3, n_s, n_b):
    mesh = plsc.VectorSubcoreMesh(
        core_axis_name="c", subcore_axis_name="s", num_cores=NC, num_subcores=NS
    )
    f = pl.kernel(
        _gather_t_body,
        out_type=jax.ShapeDtypeStruct((n_s, D, n_b), jnp.float32),
        mesh=mesh,
        scratch_types=[
            pltpu.VMEM((n_s, BLK), jnp.int32),
            pltpu.VMEM((BLK, DP), jnp.float32),
            pltpu.VMEM((BLK, DP), jnp.float32),
            pltpu.VMEM((D, BLK), jnp.float32),
            pltpu.VMEM((D, BLK), jnp.float32),
            pltpu.SemaphoreType.DMA,
            pltpu.SemaphoreType.DMA,
        ],
        compiler_params=pltpu.CompilerParams(
            use_tc_tiling_on_sc=True, needs_layout_passes=False
        ),
    )
    return f(h_pad, idx3)


def kernel(word_ids, emb, W1, b1, W2, b2, edge_index):
    B, S = word_ids.shape
    W2p = jnp.concatenate([W2, jnp.zeros((D, DP - D), jnp.float32)], axis=1)
    b2p = jnp.concatenate([b2, jnp.zeros((DP - D,), jnp.float32)])
    h_pad = _dense_transform(emb, W1, b1, W2p, b2p)
    idx3 = jnp.swapaxes(word_ids, 0, 1).reshape(S, NW, BLK).swapaxes(0, 1)
    out_t = _gather_transpose(h_pad, idx3, S, B)
    return jnp.transpose(out_t, (2, 0, 1))


# speedup vs baseline: 1.1665x; 1.1665x over previous
import functools

import jax
import jax.numpy as jnp
from jax import lax
from jax.experimental import pallas as pl
from jax.experimental.pallas import tpu as pltpu
from jax.experimental.pallas import tpu_sc as plsc

VOCAB = 100000
D = 64
DP = 128
NC = 2
NS = 16
NW = NC * NS
BLK = 128


def _dense_body(emb_ref, w1_ref, b1_ref, w2_ref, b2_ref, out_ref):
    x = emb_ref[...]
    h1 = jnp.maximum(
        jnp.dot(x, w1_ref[...], preferred_element_type=jnp.float32) + b1_ref[...],
        0.0,
    )
    out_ref[...] = (
        jnp.dot(h1, w2_ref[...], preferred_element_type=jnp.float32) + b2_ref[...]
    )


def _dense_transform(emb, W1, b1, W2p, b2p):
    rows_per_block = 2000
    grid = VOCAB // rows_per_block
    return pl.pallas_call(
        _dense_body,
        grid=(grid,),
        in_specs=[
            pl.BlockSpec((rows_per_block, D), lambda i: (i, 0)),
            pl.BlockSpec((D, D), lambda i: (0, 0)),
            pl.BlockSpec((1, D), lambda i: (0, 0)),
            pl.BlockSpec((D, DP), lambda i: (0, 0)),
            pl.BlockSpec((1, DP), lambda i: (0, 0)),
        ],
        out_specs=pl.BlockSpec((rows_per_block, DP), lambda i: (i, 0)),
        out_shape=jax.ShapeDtypeStruct((VOCAB, DP), jnp.float32),
    )(emb, W1, b1.reshape(1, D), W2p, b2p.reshape(1, DP))


def _gather_t_body(h_hbm, idx_hbm, out_hbm, idx_v, buf_a, buf_b, tr_a, tr_b,
                   gsem, wsem):
    cid = lax.axis_index("c")
    sid = lax.axis_index("s")
    w = sid * NC + cid
    n_s = idx_hbm.shape[1]
    pltpu.sync_copy(idx_hbm.at[w], idx_v)
    col = w * BLK
    iota16 = lax.iota(jnp.int32, 16)
    row_idx = [iota16 + dc * 16 for dc in range(D // 16)]
    rots = [(iota16 + t) & 15 for t in range(16)]
    bufs = ((buf_a, tr_a), (buf_b, tr_b))

    def transpose_block(src, dst):
        def jcstep(jc, carry):
            jc16 = jc * 16
            jvecs = [r + jc16 for r in rots]
            for dc in range(D // 16):
                dvec = row_idx[dc]
                for t in range(16):
                    v = plsc.load_gather(src, [jvecs[t], dvec])
                    plsc.store_scatter(dst, [dvec, jvecs[t]], v)
            return carry

        lax.fori_loop(0, BLK // 16, jcstep, 0)

    def step(i, carry):
        descs = []
        for half in range(2):
            g = 2 * i + half
            buf, tr = bufs[half]

            @pl.when(i > 0)
            def _(tr=tr):
                pltpu.make_async_copy(
                    tr, out_hbm.at[0, :, pl.ds(col, BLK)], wsem
                ).wait()

            descs.append(pltpu.async_copy(h_hbm.at[idx_v.at[g]], buf, gsem))
        for half in range(2):
            g = 2 * i + half
            buf, tr = bufs[half]
            descs[half].wait()
            transpose_block(buf, tr)
            pltpu.async_copy(tr, out_hbm.at[g, :, pl.ds(col, BLK)], wsem)
        return carry

    lax.fori_loop(0, n_s // 2, step, 0)
    pltpu.make_async_copy(tr_a, out_hbm.at[0, :, pl.ds(col, BLK)], wsem).wait()
    pltpu.make_async_copy(tr_b, out_hbm.at[0, :, pl.ds(col, BLK)], wsem).wait()


def _gather_transpose(h_pad, idx3, n_s, n_b):
    mesh = plsc.VectorSubcoreMesh(
        core_axis_name="c", subcore_axis_name="s", num_cores=NC, num_subcores=NS
    )
    f = pl.kernel(
        _gather_t_body,
        out_type=jax.ShapeDtypeStruct((n_s, D, n_b), jnp.float32),
        mesh=mesh,
        scratch_types=[
            pltpu.VMEM((n_s, BLK), jnp.int32),
            pltpu.VMEM((BLK, DP), jnp.float32),
            pltpu.VMEM((BLK, DP), jnp.float32),
            pltpu.VMEM((D, BLK), jnp.float32),
            pltpu.VMEM((D, BLK), jnp.float32),
            pltpu.SemaphoreType.DMA,
            pltpu.SemaphoreType.DMA,
        ],
        compiler_params=pltpu.CompilerParams(
            use_tc_tiling_on_sc=True, needs_layout_passes=False
        ),
    )
    return f(h_pad, idx3)


def kernel(word_ids, emb, W1, b1, W2, b2, edge_index):
    B, S = word_ids.shape
    W2p = jnp.concatenate([W2, jnp.zeros((D, DP - D), jnp.float32)], axis=1)
    b2p = jnp.concatenate([b2, jnp.zeros((DP - D,), jnp.float32)])
    h_pad = _dense_transform(emb, W1, b1, W2p, b2p)
    idx3 = jnp.swapaxes(word_ids, 0, 1).reshape(S, NW, BLK).swapaxes(0, 1)
    out_t = _gather_transpose(h_pad, idx3, S, B)
    return jnp.transpose(out_t, (2, 0, 1))


# speedup vs baseline: 1.4259x; 1.2223x over previous
import jax
import jax.numpy as jnp
from jax import lax
from jax.experimental import pallas as pl
from jax.experimental.pallas import tpu as pltpu
from jax.experimental.pallas import tpu_sc as plsc

VOCAB = 100000
D = 64
DP = 128
NC = 2
NS = 16
NW = NC * NS
BLK = 128


def _dense_body(emb_ref, w1_ref, b1_ref, w2_ref, b2_ref, out_ref):
    x = emb_ref[...]
    h1 = jnp.maximum(
        jnp.dot(x, w1_ref[...], preferred_element_type=jnp.float32) + b1_ref[...],
        0.0,
    )
    out_ref[...] = (
        jnp.dot(h1, w2_ref[...], preferred_element_type=jnp.float32) + b2_ref[...]
    )


def _dense_transform(emb, W1, b1, W2p, b2p):
    rows_per_block = 2000
    grid = VOCAB // rows_per_block
    return pl.pallas_call(
        _dense_body,
        grid=(grid,),
        in_specs=[
            pl.BlockSpec((rows_per_block, D), lambda i: (i, 0)),
            pl.BlockSpec((D, D), lambda i: (0, 0)),
            pl.BlockSpec((1, D), lambda i: (0, 0)),
            pl.BlockSpec((D, DP), lambda i: (0, 0)),
            pl.BlockSpec((1, DP), lambda i: (0, 0)),
        ],
        out_specs=pl.BlockSpec((rows_per_block, DP), lambda i: (i, 0)),
        out_shape=jax.ShapeDtypeStruct((VOCAB, DP), jnp.float32),
    )(emb, W1, b1.reshape(1, D), W2p, b2p.reshape(1, DP))


NSLOT = 4


def _gather_t_body(h_hbm, idx_hbm, out_hbm, idx_v,
                   buf_0, buf_1, buf_2, buf_3, tr_0, tr_1, tr_2, tr_3,
                   gsem_0, gsem_1, gsem_2, gsem_3, wsem):
    cid = lax.axis_index("c")
    sid = lax.axis_index("s")
    w = sid * NC + cid
    n_s = idx_hbm.shape[1]
    pltpu.sync_copy(idx_hbm.at[w], idx_v)
    col = w * BLK
    iota16 = lax.iota(jnp.int32, 16)
    row_idx = [iota16 + dc * 16 for dc in range(D // 16)]
    rots = [(iota16 + t) & 15 for t in range(16)]
    bufs = ((buf_0, tr_0, gsem_0), (buf_1, tr_1, gsem_1),
            (buf_2, tr_2, gsem_2), (buf_3, tr_3, gsem_3))

    def transpose_block(src, dst):
        def jcstep(jc, carry):
            jc16 = jc * 16
            jvecs = [r + jc16 for r in rots]
            for dc in range(D // 16):
                dvec = row_idx[dc]
                for t in range(16):
                    v = plsc.load_gather(src, [jvecs[t], dvec])
                    plsc.store_scatter(dst, [dvec, jvecs[t]], v)
            return carry

        lax.fori_loop(0, BLK // 16, jcstep, 0)

    n_iter = n_s // NSLOT

    for k in range(NSLOT):
        buf, _tr, gsem = bufs[k]
        pltpu.async_copy(h_hbm.at[idx_v.at[k]], buf, gsem)

    def step(i, carry):
        for k in range(NSLOT):
            g = NSLOT * i + k
            buf, tr, gsem = bufs[k]
            pltpu.make_async_copy(h_hbm.at[idx_v.at[g]], buf, gsem).wait()

            @pl.when(i > 0)
            def _(tr=tr):
                pltpu.make_async_copy(
                    tr, out_hbm.at[0, :, pl.ds(col, BLK)], wsem
                ).wait()

            transpose_block(buf, tr)
            @pl.when(i < n_iter - 1)
            def _(buf=buf, gsem=gsem, g=g):
                pltpu.async_copy(h_hbm.at[idx_v.at[g + NSLOT]], buf, gsem)

            pltpu.async_copy(tr, out_hbm.at[g, :, pl.ds(col, BLK)], wsem)
        return carry

    lax.fori_loop(0, n_iter, step, 0)
    for k in range(NSLOT):
        _buf, tr, _gsem = bufs[k]
        pltpu.make_async_copy(tr, out_hbm.at[0, :, pl.ds(col, BLK)], wsem).wait()


def _gather_transpose(h_pad, idx3, n_s, n_b):
    mesh = plsc.VectorSubcoreMesh(
        core_axis_name="c", subcore_axis_name="s", num_cores=NC, num_subcores=NS
    )
    f = pl.kernel(
        _gather_t_body,
        out_type=jax.ShapeDtypeStruct((n_s, D, n_b), jnp.float32),
        mesh=mesh,
        scratch_types=(
            [pltpu.VMEM((n_s, BLK), jnp.int32)]
            + [pltpu.VMEM((BLK, DP), jnp.float32)] * NSLOT
            + [pltpu.VMEM((D, BLK), jnp.float32)] * NSLOT
            + [pltpu.SemaphoreType.DMA] * (NSLOT + 1)
        ),
        compiler_params=pltpu.CompilerParams(
            use_tc_tiling_on_sc=True, needs_layout_passes=False
        ),
    )
    return f(h_pad, idx3)


def kernel(word_ids, emb, W1, b1, W2, b2, edge_index):
    B, S = word_ids.shape
    W2p = jnp.concatenate([W2, jnp.zeros((D, DP - D), jnp.float32)], axis=1)
    b2p = jnp.concatenate([b2, jnp.zeros((DP - D,), jnp.float32)])
    h_pad = _dense_transform(emb, W1, b1, W2p, b2p)
    idx3 = jnp.swapaxes(word_ids, 0, 1).reshape(S, NW, BLK).swapaxes(0, 1)
    out_t = _gather_transpose(h_pad, idx3, S, B)
    return jnp.transpose(out_t, (2, 0, 1))
